# baseline (speedup 1.0000x reference)
import functools

import jax
import jax.numpy as jnp
from jax import lax
from jax.experimental import pallas as pl
from jax.experimental.pallas import tpu as pltpu
from jax.experimental.pallas import tpu_sc as plsc

_VOCAB = 2048
_DIM = 2048
_SEQ = 50
_NCLASS = 20
_CW = 20
_STRIDE = 21
_ROWS = _VOCAB + 8
_NC = 2
_NS = 16
_NW = _NC * _NS
_L = 16


def _project_body(table_ref, w1_ref, w2_ref, b1_ref, b2_ref, out_ref):
    t = table_ref[...]
    h = lax.dot_general(t, w1_ref[...], (((1,), (1,)), ((), ())),
                        preferred_element_type=jnp.float32)
    proj = lax.dot_general(h, w2_ref[...], (((1,), (1,)), ((), ())),
                           preferred_element_type=jnp.float32)
    out_ref[0:_VOCAB, 0:_CW] = proj * (1.0 / _SEQ)
    brow = lax.dot_general(b1_ref[...], w2_ref[...], (((1,), (1,)), ((), ())),
                           preferred_element_type=jnp.float32) + b2_ref[...]
    out_ref[_VOCAB:_ROWS, 0:_CW] = jnp.broadcast_to(brow, (_ROWS - _VOCAB, _CW))


def _project_table(table, w1, w2, b1, b2):
    return pl.pallas_call(
        _project_body,
        out_shape=jax.ShapeDtypeStruct((_ROWS, _STRIDE), jnp.float32),
    )(table, w1, w2, b1.reshape(1, -1), b2.reshape(1, -1))


def _sc_pool(tblp_flat, x2, s, bpw):
    mesh = plsc.VectorSubcoreMesh(core_axis_name="c", subcore_axis_name="s")
    groups = bpw // _L

    @functools.partial(
        pl.kernel,
        mesh=mesh,
        out_type=jax.ShapeDtypeStruct((_NW, bpw * _CW), jnp.float32),
        compiler_params=pltpu.CompilerParams(needs_layout_passes=False),
        scratch_types=[
            pltpu.VMEM((_ROWS * _STRIDE,), jnp.float32),
            pltpu.VMEM((bpw * s,), jnp.int32),
            pltpu.VMEM((bpw * _CW,), jnp.float32),
        ],
    )
    def pool(tbl_hbm, x_hbm, out_hbm, tbl_v, idx_v, outb_v):
        wid = lax.axis_index("s") * _NC + lax.axis_index("c")
        pltpu.sync_copy(tbl_hbm, tbl_v)
        pltpu.sync_copy(x_hbm.at[wid], idx_v)
        lane = jnp.arange(_L, dtype=jnp.int32)
        lane_s = lane * s
        lane_c = lane * _CW
        bias_base = _VOCAB * _STRIDE
        init = tuple(
            plsc.load_gather(tbl_v, [jnp.full((_L,), bias_base + c, jnp.int32)])
            for c in range(_CW))
        for g in range(groups):
            def body(i, carry, _g=g):
                rows = plsc.load_gather(idx_v, [lane_s + (_g * (_L * s) + i)])
                base = rows * _STRIDE
                return tuple(carry[c] + plsc.load_gather(tbl_v, [base + c])
                             for c in range(_CW))
            acc = lax.fori_loop(0, s, body, init)
            for c in range(_CW):
                plsc.store_scatter(outb_v, [lane_c + (g * (_L * _CW) + c)], acc[c])
        pltpu.sync_copy(outb_v, out_hbm.at[wid])

    return pool(tblp_flat, x2)


def kernel(x, table, W1, b1, W2, b2):
    b, s = x.shape
    bpw = b // _NW
    tblp = _project_table(table, W1, W2, b1, b2)
    outw = _sc_pool(tblp.reshape(-1), x.reshape(_NW, bpw * s), s, bpw)
    return outw.reshape(b, _CW)
